# Initial kernel scaffold; baseline (speedup 1.0000x reference)
import jax
import jax.numpy as jnp
from jax.experimental import pallas as pl


def kernel(x, thetas_even, thetas_odd):
    raise NotImplementedError("write your pallas kernel here")


# speedup vs baseline: 10.2174x; 10.2174x over previous
import jax
import jax.numpy as jnp
from jax import lax
from jax.experimental import pallas as pl
from jax.experimental.pallas import tpu as pltpu
from jax.experimental.pallas import tpu_sc as plsc

DIM = 256
HALF = DIM // 2
PAD = 144
BATCH = 4096
NPAIRS = 64
NW = 32
ROWS_PER_W = BATCH // NW
L = 16
NG = HALF // L
UNROLL = 2


def _trig_body(te_ref, top_ref, ce_ref, se_ref, co_ref, so_ref):
    ce_ref[...] = jnp.cos(2.0 * te_ref[...])
    se_ref[...] = jnp.sin(2.0 * te_ref[...])
    co_ref[...] = jnp.cos(2.0 * top_ref[...])
    so_ref[...] = jnp.sin(2.0 * top_ref[...])


_trig = pl.pallas_call(
    _trig_body,
    out_shape=[jax.ShapeDtypeStruct((NPAIRS, HALF), jnp.float32)] * 4,
)


def _take(v, idx):
    return lax.gather(
        v, idx[:, None],
        lax.GatherDimensionNumbers(offset_dims=(), collapsed_slice_dims=(0,),
                                   start_index_map=(0,)),
        slice_sizes=(1,),
        mode=lax.GatherScatterMode.PROMISE_IN_BOUNDS)


def _clements_body(x_hbm, ce_hbm, se_hbm, co_hbm, so_hbm, out_hbm,
                   X, A, B, CE, SE, CO, SO):
    wid = lax.axis_index("s") * 2 + lax.axis_index("c")
    base = wid * ROWS_PER_W
    pltpu.sync_copy(x_hbm.at[pl.ds(base, ROWS_PER_W)], X)
    pltpu.sync_copy(ce_hbm, CE)
    pltpu.sync_copy(se_hbm, SE)
    pltpu.sync_copy(co_hbm, CO)
    pltpu.sync_copy(so_hbm, SO)

    iota = lax.iota(jnp.int32, L)
    idx_a = (2 * iota) & 15
    idx_b = (2 * iota + 1) & 15
    lo_half = iota < 8
    zeros = jnp.zeros((L,), jnp.float32)

    @plsc.parallel_loop(0, ROWS_PER_W, unroll=UNROLL)
    def _deinterleave(r):
        for j in range(NG):
            v0 = X[r, pl.ds(32 * j, L)]
            v1 = X[r, pl.ds(32 * j + 16, L)]
            A[r, pl.ds(L * j, L)] = jnp.where(
                lo_half, _take(v0, idx_a), _take(v1, idx_a))
            B[r, pl.ds(L * j, L)] = jnp.where(
                lo_half, _take(v0, idx_b), _take(v1, idx_b))
        A[r, pl.ds(HALF, L)] = zeros

    def layer_pair(l, carry):
        ces = [CE[l, pl.ds(L * g, L)] for g in range(NG)]
        ses = [SE[l, pl.ds(L * g, L)] for g in range(NG)]

        @plsc.parallel_loop(0, ROWS_PER_W, unroll=UNROLL)
        def _even(r):
            for g in range(NG):
                sl = pl.ds(L * g, L)
                a = A[r, sl]
                b = B[r, sl]
                A[r, sl] = a * ces[g] + b * ses[g]
                B[r, sl] = a * ses[g] - b * ces[g]

        cos_ = [CO[l, pl.ds(L * g, L)] for g in range(NG)]
        sos = [SO[l, pl.ds(L * g, L)] for g in range(NG)]

        @plsc.parallel_loop(0, ROWS_PER_W, unroll=UNROLL)
        def _odd(r):
            for g in range(NG):
                sl = pl.ds(L * g, L)
                sl1 = pl.ds(L * g + 1, L)
                b = B[r, sl]
                a1 = A[r, sl1]
                B[r, sl] = b * cos_[g] + a1 * sos[g]
                A[r, sl1] = b * sos[g] - a1 * cos_[g]

        return carry

    lax.fori_loop(0, NPAIRS, layer_pair, 0)

    idx_h = iota >> 1
    even_lane = (iota & 1) == 0

    @plsc.parallel_loop(0, ROWS_PER_W, unroll=UNROLL)
    def _interleave(r):
        for j in range(NG):
            a = A[r, pl.ds(L * j, L)]
            b = B[r, pl.ds(L * j, L)]
            X[r, pl.ds(32 * j, L)] = jnp.where(
                even_lane, _take(a, idx_h), _take(b, idx_h))
            X[r, pl.ds(32 * j + 16, L)] = jnp.where(
                even_lane, _take(a, 8 + idx_h), _take(b, 8 + idx_h))

    pltpu.sync_copy(X, out_hbm.at[pl.ds(base, ROWS_PER_W)])


_clements_sc = pl.kernel(
    _clements_body,
    out_type=jax.ShapeDtypeStruct((BATCH, DIM), jnp.float32),
    mesh=plsc.VectorSubcoreMesh(core_axis_name="c", subcore_axis_name="s",
                                num_cores=2, num_subcores=16),
    compiler_params=pltpu.CompilerParams(use_tc_tiling_on_sc=False),
    scratch_types=[
        pltpu.VMEM((ROWS_PER_W, DIM), jnp.float32),
        pltpu.VMEM((ROWS_PER_W, PAD), jnp.float32),
        pltpu.VMEM((ROWS_PER_W, HALF), jnp.float32),
        pltpu.VMEM((NPAIRS, HALF), jnp.float32),
        pltpu.VMEM((NPAIRS, HALF), jnp.float32),
        pltpu.VMEM((NPAIRS, HALF), jnp.float32),
        pltpu.VMEM((NPAIRS, HALF), jnp.float32),
    ],
)


def kernel(x, thetas_even, thetas_odd):
    to_p = jnp.pad(thetas_odd, ((0, 0), (0, 1)))
    ce, se, co, so = _trig(thetas_even, to_p)
    return _clements_sc(x, ce, se, co, so)
